# Initial kernel scaffold; baseline (speedup 1.0000x reference)
import jax
import jax.numpy as jnp
from jax.experimental import pallas as pl


def kernel(x, A, W, temperature, q):
    raise NotImplementedError("write your pallas kernel here")


# speedup vs baseline: 8.7248x; 8.7248x over previous
import functools

import jax
import jax.numpy as jnp
from jax import lax
from jax.experimental import pallas as pl
from jax.experimental.pallas import tpu as pltpu

B, N, DF, K = 4, 2048, 128, 16
RB = 256
NB = N // RB

NEG = float("-inf")


def _embed_body(x_ref, w_ref, xe_ref, xc_ref, xct_ref):
    x = x_ref[0]
    w = w_ref[...]
    xe = jnp.dot(x, w, preferred_element_type=jnp.float32)
    xe_ref[0] = xe
    mean = jnp.mean(xe, axis=0, keepdims=True)
    xc = xe - mean
    xc_ref[0] = xc
    xct_ref[0] = xc.T


def _topk_body(scale_ref, xcr_ref, xct_ref, q_ref, lp_ref, idx_ref):
    r0 = pl.program_id(1) * RB
    xr = xcr_ref[0]
    xt = xct_ref[0]
    s = jnp.dot(xr, xt, preferred_element_type=jnp.float32)
    x2r = jnp.sum(xr * xr, axis=1, keepdims=True)
    x2c = jnp.sum(xt * xt, axis=0, keepdims=True)
    d = jnp.maximum(x2r + x2c - 2.0 * s, 0.0)
    scale = scale_ref[0]
    col = lax.broadcasted_iota(jnp.int32, (RB, N), 1)
    row = lax.broadcasted_iota(jnp.int32, (RB, N), 0) + r0
    g = jnp.log(-jnp.log(q_ref[0]))
    vals = jnp.where(col == row, NEG, -d * scale - g)
    for t in range(K):
        m = jnp.max(vals, axis=1, keepdims=True)
        eq = vals == m
        sel = jnp.min(jnp.where(eq, col, N), axis=1, keepdims=True)
        lp_ref[0, :, t : t + 1] = m
        idx_ref[0, :, t : t + 1] = sel
        vals = jnp.where(col == sel, NEG, vals)


@jax.jit
def kernel(x, A, W, temperature, q):
    del A
    scale = jnp.exp(jnp.clip(temperature, -4.0, 4.0)).reshape(1)

    xe, xc, xct = pl.pallas_call(
        _embed_body,
        grid=(B,),
        in_specs=[
            pl.BlockSpec((1, N, DF), lambda b: (b, 0, 0)),
            pl.BlockSpec((DF, DF), lambda b: (0, 0)),
        ],
        out_specs=[
            pl.BlockSpec((1, N, DF), lambda b: (b, 0, 0)),
            pl.BlockSpec((1, N, DF), lambda b: (b, 0, 0)),
            pl.BlockSpec((1, DF, N), lambda b: (b, 0, 0)),
        ],
        out_shape=[
            jax.ShapeDtypeStruct((B, N, DF), jnp.float32),
            jax.ShapeDtypeStruct((B, N, DF), jnp.float32),
            jax.ShapeDtypeStruct((B, DF, N), jnp.float32),
        ],
    )(x, W)

    lp, idx = pl.pallas_call(
        _topk_body,
        grid=(B, NB),
        in_specs=[
            pl.BlockSpec(memory_space=pltpu.SMEM),
            pl.BlockSpec((1, RB, DF), lambda b, r: (b, r, 0)),
            pl.BlockSpec((1, DF, N), lambda b, r: (b, 0, 0)),
            pl.BlockSpec((1, RB, N), lambda b, r: (b, r, 0)),
        ],
        out_specs=[
            pl.BlockSpec((1, RB, K), lambda b, r: (b, r, 0)),
            pl.BlockSpec((1, RB, K), lambda b, r: (b, r, 0)),
        ],
        out_shape=[
            jax.ShapeDtypeStruct((B, N, K), jnp.float32),
            jax.ShapeDtypeStruct((B, N, K), jnp.int32),
        ],
    )(scale, xc, xct, q)

    offs = (jnp.arange(B, dtype=jnp.int32) * N)[:, None]
    e0 = idx.reshape(B, N * K) + offs
    e1 = jnp.repeat(jnp.arange(N, dtype=jnp.int32), K)[None, :] + offs
    edges_sparse = jnp.stack((e0, e1), 0).reshape(2, -1)
    return xe, edges_sparse, lp


# speedup vs baseline: 9.4104x; 1.0786x over previous
import functools

import jax
import jax.numpy as jnp
from jax import lax
from jax.experimental import pallas as pl
from jax.experimental.pallas import tpu as pltpu
from jax.experimental.pallas import tpu_sc as plsc

B, N, DF, K = 4, 2048, 128, 16
RB = 256
NB = N // RB
BN = B * N

NW = 32
ROWS_W = BN // NW
RG = 16
NGRP = ROWS_W // RG
GCAP = 32
CCAP = 128

NEG = float("-inf")


def _embed_body(x_ref, w_ref, xe_ref, xc_ref, xct_ref):
    x = x_ref[0]
    w = w_ref[...]
    xe = jnp.dot(x, w, preferred_element_type=jnp.float32)
    xe_ref[0] = xe
    mean = jnp.mean(xe, axis=0, keepdims=True)
    xc = xe - mean
    xc_ref[0] = xc
    xct_ref[0] = xc.T


def _lq_body(scale_ref, xcr_ref, xct_ref, q_ref, lq_ref, gmax_ref, tau_ref):
    r0 = pl.program_id(1) * RB
    xr = xcr_ref[0]
    xt = xct_ref[0]
    s = jnp.dot(xr, xt, preferred_element_type=jnp.float32)
    x2r = jnp.sum(xr * xr, axis=1, keepdims=True)
    x2c = jnp.sum(xt * xt, axis=0, keepdims=True)
    d = jnp.maximum(x2r + x2c - 2.0 * s, 0.0)
    scale = scale_ref[0]
    col = lax.broadcasted_iota(jnp.int32, (RB, N), 1)
    row = lax.broadcasted_iota(jnp.int32, (RB, N), 0) + r0
    g = jnp.log(-jnp.log(q_ref[0]))
    vals = jnp.where(col == row, NEG, -d * scale - g)
    lq_ref[0] = vals
    gm = vals[:, 0:128]
    for t in range(1, 16):
        gm = jnp.maximum(gm, vals[:, t * 128 : (t + 1) * 128])
    gmax_ref[0] = gm
    g2 = gm
    for _ in range(15):
        m = jnp.max(g2, axis=1, keepdims=True)
        g2 = jnp.where(g2 == m, NEG, g2)
    tau_ref[0] = jnp.max(g2, axis=1, keepdims=True)


def _sc_topk(lq_hbm, gmax_hbm, tau_hbm, lp_hbm, idx_hbm,
             rowb, gb, tb, lpb, idxb, gidb, candv, candp):
    wid = lax.axis_index("s") * 2 + lax.axis_index("c")
    wrow0 = wid * ROWS_W
    lane = lax.iota(jnp.int32, 16)
    ninf = jnp.full((16,), NEG, jnp.float32)

    def merge_body(m, carry):
        rv, ri, ccs = carry
        valid = (lane + m * 16) < ccs
        c = jnp.where(valid, candv[pl.ds(m * 16, 16)], ninf)
        p = candp[pl.ds(m * 16, 16)]
        cs, cp = plsc.sort_key_val(c, p, descending=True)
        csr = jnp.flip(cs, 0)
        cpr = jnp.flip(cp, 0)
        take = rv >= csr
        nv = jnp.where(take, rv, csr)
        np_ = jnp.where(take, ri, cpr)
        rv, ri = plsc.sort_key_val(nv, np_, descending=True)
        return rv, ri, ccs

    def row_body(i, _):
        tsplat = plsc.load_gather(tb, [jnp.full((16,), i, jnp.int32)])
        def s1(j, cur):
            gm = gb[pl.ds(i * 128 + j * 16, 16)]
            msk = gm >= tsplat
            pos = cur + plsc.cumsum(msk.astype(jnp.int32)) - 1
            msk = msk & (pos < GCAP)
            plsc.store_scatter(gidb, [pos], lane + j * 16, mask=msk)
            return cur + plsc.all_reduce_population_count(msk)
        gcnt = lax.fori_loop(0, 8, s1, jnp.zeros((16,), jnp.int32))

        def round_fn(r, ccur):
            gids = gidb[pl.ds(r * 16, 16)]
            gvalid = (lane + r * 16) < gcnt
            gids = jnp.where(gvalid, gids, 0)
            def s2(t, cc):
                colidx = gids + t * 128
                v = plsc.load_gather(rowb, [colidx + i * N], mask=gvalid)
                msk = gvalid & (v >= tsplat)
                pos = cc + plsc.cumsum(msk.astype(jnp.int32)) - 1
                msk = msk & (pos < CCAP)
                plsc.store_scatter(candv, [pos], v, mask=msk)
                plsc.store_scatter(candp, [pos], colidx, mask=msk)
                return cc + plsc.all_reduce_population_count(msk)
            return lax.fori_loop(0, 16, s2, ccur)

        ccur = round_fn(0, jnp.zeros((16,), jnp.int32))
        gs = jnp.max(gcnt)
        ccur = lax.cond(gs > 16, lambda c: round_fn(1, c), lambda c: c, ccur)

        cs_scalar = jnp.max(ccur)
        nmerge = (cs_scalar + 15) // 16
        rv0 = ninf
        ri0 = jnp.zeros((16,), jnp.int32)
        rv, ri, _ = lax.fori_loop(0, nmerge, merge_body, (rv0, ri0, ccur))
        lpb[pl.ds(i * 16, 16)] = rv
        idxb[pl.ds(i * 16, 16)] = ri
        return 0

    def group_body(gidx, _):
        row0 = wrow0 + gidx * RG
        pltpu.sync_copy(lq_hbm.at[pl.ds(row0 * N, RG * N)], rowb)
        pltpu.sync_copy(gmax_hbm.at[pl.ds(row0 * 128, RG * 128)], gb)
        pltpu.sync_copy(tau_hbm.at[pl.ds(row0, RG)], tb)
        lax.fori_loop(0, RG, row_body, 0)
        pltpu.sync_copy(lpb, lp_hbm.at[pl.ds(row0 * K, RG * K)])
        pltpu.sync_copy(idxb, idx_hbm.at[pl.ds(row0 * K, RG * K)])
        return 0

    lax.fori_loop(0, NGRP, group_body, 0)


_sc_topk_call = functools.partial(
    pl.kernel,
    out_type=[
        jax.ShapeDtypeStruct((BN * K,), jnp.float32),
        jax.ShapeDtypeStruct((BN * K,), jnp.int32),
    ],
    mesh=plsc.VectorSubcoreMesh(core_axis_name="c", subcore_axis_name="s"),
    compiler_params=pltpu.CompilerParams(needs_layout_passes=False),
    scratch_types=[
        pltpu.VMEM((RG * N,), jnp.float32),
        pltpu.VMEM((RG * 128,), jnp.float32),
        pltpu.VMEM((RG,), jnp.float32),
        pltpu.VMEM((RG * K,), jnp.float32),
        pltpu.VMEM((RG * K,), jnp.int32),
        pltpu.VMEM((GCAP,), jnp.int32),
        pltpu.VMEM((CCAP,), jnp.float32),
        pltpu.VMEM((CCAP,), jnp.int32),
    ],
)(_sc_topk)


@jax.jit
def kernel(x, A, W, temperature, q):
    del A
    scale = jnp.exp(jnp.clip(temperature, -4.0, 4.0)).reshape(1)

    xe, xc, xct = pl.pallas_call(
        _embed_body,
        grid=(B,),
        in_specs=[
            pl.BlockSpec((1, N, DF), lambda b: (b, 0, 0)),
            pl.BlockSpec((DF, DF), lambda b: (0, 0)),
        ],
        out_specs=[
            pl.BlockSpec((1, N, DF), lambda b: (b, 0, 0)),
            pl.BlockSpec((1, N, DF), lambda b: (b, 0, 0)),
            pl.BlockSpec((1, DF, N), lambda b: (b, 0, 0)),
        ],
        out_shape=[
            jax.ShapeDtypeStruct((B, N, DF), jnp.float32),
            jax.ShapeDtypeStruct((B, N, DF), jnp.float32),
            jax.ShapeDtypeStruct((B, DF, N), jnp.float32),
        ],
    )(x, W)

    lq, gmax, tau = pl.pallas_call(
        _lq_body,
        grid=(B, NB),
        in_specs=[
            pl.BlockSpec(memory_space=pltpu.SMEM),
            pl.BlockSpec((1, RB, DF), lambda b, r: (b, r, 0)),
            pl.BlockSpec((1, DF, N), lambda b, r: (b, 0, 0)),
            pl.BlockSpec((1, RB, N), lambda b, r: (b, r, 0)),
        ],
        out_specs=[
            pl.BlockSpec((1, RB, N), lambda b, r: (b, r, 0)),
            pl.BlockSpec((1, RB, 128), lambda b, r: (b, r, 0)),
            pl.BlockSpec((1, RB, 1), lambda b, r: (b * NB + r, 0, 0)),
        ],
        out_shape=[
            jax.ShapeDtypeStruct((B, N, N), jnp.float32),
            jax.ShapeDtypeStruct((B, N, 128), jnp.float32),
            jax.ShapeDtypeStruct((B * NB, RB, 1), jnp.float32),
        ],
    )(scale, xc, xct, q)

    lp_flat, idx_flat = _sc_topk_call(
        lq.reshape(-1), gmax.reshape(-1), tau.reshape(-1)
    )
    lp = lp_flat.reshape(B, N, K)
    idx = idx_flat.reshape(B, N, K)

    offs = (jnp.arange(B, dtype=jnp.int32) * N)[:, None]
    e0 = idx.reshape(B, N * K) + offs
    e1 = jnp.repeat(jnp.arange(N, dtype=jnp.int32), K)[None, :] + offs
    edges_sparse = jnp.stack((e0, e1), 0).reshape(2, -1)
    return xe, edges_sparse, lp


# speedup vs baseline: 13.5721x; 1.4422x over previous
import functools

import jax
import jax.numpy as jnp
from jax import lax
from jax.experimental import pallas as pl
from jax.experimental.pallas import tpu as pltpu
from jax.experimental.pallas import tpu_sc as plsc

B, N, DF, K = 4, 2048, 128, 16
RB = 256
NB = N // RB
BN = B * N

NW = 32
ROWS_W = BN // NW
RG = 16
NGRP = ROWS_W // RG
GCAP = 32
CCAP = 128

NEG = float("-inf")


def _embed_body(x_ref, w_ref, xe_ref, xc_ref, xct_ref):
    x = x_ref[0]
    w = w_ref[...]
    xe = jnp.dot(x, w, preferred_element_type=jnp.float32)
    xe_ref[0] = xe
    mean = jnp.mean(xe, axis=0, keepdims=True)
    xc = xe - mean
    xc_ref[0] = xc
    xct_ref[0] = xc.T


def _lq_body(scale_ref, xcr_ref, xct_ref, q_ref, lq_ref, gmax_ref, tau_ref):
    r0 = pl.program_id(1) * RB
    xr = xcr_ref[0]
    xt = xct_ref[0]
    s = jnp.dot(xr, xt, preferred_element_type=jnp.float32)
    x2r = jnp.sum(xr * xr, axis=1, keepdims=True)
    x2c = jnp.sum(xt * xt, axis=0, keepdims=True)
    d = jnp.maximum(x2r + x2c - 2.0 * s, 0.0)
    scale = scale_ref[0]
    col = lax.broadcasted_iota(jnp.int32, (RB, N), 1)
    row = lax.broadcasted_iota(jnp.int32, (RB, N), 0) + r0
    g = jnp.log(-jnp.log(q_ref[0]))
    vals = jnp.where(col == row, NEG, -d * scale - g)
    lq_ref[0] = vals
    gm = vals[:, 0:128]
    for t in range(1, 16):
        gm = jnp.maximum(gm, vals[:, t * 128 : (t + 1) * 128])
    gmax_ref[0] = gm
    g2 = gm
    for _ in range(15):
        m = jnp.max(g2, axis=1, keepdims=True)
        g2 = jnp.where(g2 == m, NEG, g2)
    tau_ref[0] = jnp.max(g2, axis=1, keepdims=True)


def _sc_topk(lq_hbm, gmax_hbm, tau_hbm, lp_hbm, idx_hbm,
             rowbs, gbs, tbs, lpbs, idxbs, gidb, candv, candp,
             insems, outsems):
    wid = lax.axis_index("s") * 2 + lax.axis_index("c")
    wrow0 = wid * ROWS_W
    lane = lax.iota(jnp.int32, 16)
    ninf = jnp.full((16,), NEG, jnp.float32)

    def fetch(gidx, p):
        row0 = wrow0 + gidx * RG
        pltpu.async_copy(lq_hbm.at[pl.ds(row0 * N, RG * N)], rowbs[p],
                         insems[p])
        pltpu.async_copy(gmax_hbm.at[pl.ds(row0 * 128, RG * 128)], gbs[p],
                         insems[p])
        pltpu.async_copy(tau_hbm.at[pl.ds(row0, RG)], tbs[p], insems[p])

    def drain_in(gidx, p):
        row0 = wrow0 + gidx * RG
        pltpu.make_async_copy(lq_hbm.at[pl.ds(row0 * N, RG * N)], rowbs[p],
                              insems[p]).wait()
        pltpu.make_async_copy(gmax_hbm.at[pl.ds(row0 * 128, RG * 128)],
                              gbs[p], insems[p]).wait()
        pltpu.make_async_copy(tau_hbm.at[pl.ds(row0, RG)], tbs[p],
                              insems[p]).wait()

    def merge_body(m, carry):
        rv, ri, ccs = carry
        valid = (lane + m * 16) < ccs
        c = jnp.where(valid, candv[pl.ds(m * 16, 16)], ninf)
        p = candp[pl.ds(m * 16, 16)]
        cs, cp = plsc.sort_key_val(c, p, descending=True)
        csr = jnp.flip(cs, 0)
        cpr = jnp.flip(cp, 0)
        take = rv >= csr
        nv = jnp.where(take, rv, csr)
        np_ = jnp.where(take, ri, cpr)
        rv, ri = plsc.sort_key_val(nv, np_, descending=True)
        return rv, ri, ccs

    def process(p):
        rowb, gb, tb, lpb, idxb = rowbs[p], gbs[p], tbs[p], lpbs[p], idxbs[p]

        def row_body(i, _):
            tsplat = plsc.load_gather(tb, [jnp.full((16,), i, jnp.int32)])

            @plsc.parallel_loop(0, 8, 1, unroll=4,
                                carry=jnp.zeros((16,), jnp.int32))
            def gcnt(j, cur):
                gm = gb[pl.ds(i * 128 + j * 16, 16)]
                msk = gm >= tsplat
                pos = cur + plsc.cumsum(msk.astype(jnp.int32)) - 1
                msk = msk & (pos < GCAP)
                plsc.store_scatter(gidb, [pos], lane + j * 16, mask=msk)
                return cur + plsc.all_reduce_population_count(msk)

            def round_fn(r, ccur):
                gids = gidb[pl.ds(r * 16, 16)]
                gvalid = (lane + r * 16) < gcnt
                gids = jnp.where(gvalid, gids, 0)

                @plsc.parallel_loop(0, 16, 1, unroll=4, carry=ccur)
                def cc_out(t, cc):
                    colidx = gids + t * 128
                    v = plsc.load_gather(rowb, [colidx + i * N], mask=gvalid)
                    msk = gvalid & (v >= tsplat)
                    pos = cc + plsc.cumsum(msk.astype(jnp.int32)) - 1
                    msk = msk & (pos < CCAP)
                    plsc.store_scatter(candv, [pos], v, mask=msk)
                    plsc.store_scatter(candp, [pos], colidx, mask=msk)
                    return cc + plsc.all_reduce_population_count(msk)
                return cc_out

            ccur = round_fn(0, jnp.zeros((16,), jnp.int32))
            gs = jnp.max(gcnt)
            ccur = lax.cond(gs > 16, lambda c: round_fn(1, c),
                            lambda c: c, ccur)

            nmerge = (jnp.max(ccur) + 15) // 16
            rv, ri, _ = lax.fori_loop(
                0, nmerge, merge_body,
                (ninf, jnp.zeros((16,), jnp.int32), ccur))
            lpb[pl.ds(i * 16, 16)] = rv
            idxb[pl.ds(i * 16, 16)] = ri
            return 0

        lax.fori_loop(0, RG, row_body, 0)

    def put(gidx, p):
        row0 = wrow0 + gidx * RG
        pltpu.async_copy(lpbs[p], lp_hbm.at[pl.ds(row0 * K, RG * K)],
                         outsems[p])
        pltpu.async_copy(idxbs[p], idx_hbm.at[pl.ds(row0 * K, RG * K)],
                         outsems[p])

    def drain_out(gidx, p):
        row0 = wrow0 + gidx * RG
        pltpu.make_async_copy(lpbs[p], lp_hbm.at[pl.ds(row0 * K, RG * K)],
                              outsems[p]).wait()
        pltpu.make_async_copy(idxbs[p], idx_hbm.at[pl.ds(row0 * K, RG * K)],
                              outsems[p]).wait()

    fetch(0, 0)

    def group_pair(g2, _):
        g0 = g2 * 2
        for p in range(2):
            g = g0 + p
            drain_in(g, p)
            nxt = jnp.minimum(g + 1, NGRP - 1)

            @pl.when(g + 1 < NGRP)
            def _():
                fetch(nxt, 1 - p)

            @pl.when(g >= 2)
            def _():
                drain_out(g - 2, p)

            process(p)
            put(g, p)
        return 0

    lax.fori_loop(0, NGRP // 2, group_pair, 0)
    drain_out(NGRP - 2, 0)
    drain_out(NGRP - 1, 1)


_sc_topk_call = functools.partial(
    pl.kernel,
    out_type=[
        jax.ShapeDtypeStruct((BN * K,), jnp.float32),
        jax.ShapeDtypeStruct((BN * K,), jnp.int32),
    ],
    mesh=plsc.VectorSubcoreMesh(core_axis_name="c", subcore_axis_name="s"),
    compiler_params=pltpu.CompilerParams(needs_layout_passes=False),
    scratch_types=[
        [pltpu.VMEM((RG * N,), jnp.float32)] * 2,
        [pltpu.VMEM((RG * 128,), jnp.float32)] * 2,
        [pltpu.VMEM((RG,), jnp.float32)] * 2,
        [pltpu.VMEM((RG * K,), jnp.float32)] * 2,
        [pltpu.VMEM((RG * K,), jnp.int32)] * 2,
        pltpu.VMEM((GCAP,), jnp.int32),
        pltpu.VMEM((CCAP,), jnp.float32),
        pltpu.VMEM((CCAP,), jnp.int32),
        [pltpu.SemaphoreType.DMA] * 2,
        [pltpu.SemaphoreType.DMA] * 2,
    ],
)(_sc_topk)


@jax.jit
def kernel(x, A, W, temperature, q):
    del A
    scale = jnp.exp(jnp.clip(temperature, -4.0, 4.0)).reshape(1)

    xe, xc, xct = pl.pallas_call(
        _embed_body,
        grid=(B,),
        in_specs=[
            pl.BlockSpec((1, N, DF), lambda b: (b, 0, 0)),
            pl.BlockSpec((DF, DF), lambda b: (0, 0)),
        ],
        out_specs=[
            pl.BlockSpec((1, N, DF), lambda b: (b, 0, 0)),
            pl.BlockSpec((1, N, DF), lambda b: (b, 0, 0)),
            pl.BlockSpec((1, DF, N), lambda b: (b, 0, 0)),
        ],
        out_shape=[
            jax.ShapeDtypeStruct((B, N, DF), jnp.float32),
            jax.ShapeDtypeStruct((B, N, DF), jnp.float32),
            jax.ShapeDtypeStruct((B, DF, N), jnp.float32),
        ],
    )(x, W)

    lq, gmax, tau = pl.pallas_call(
        _lq_body,
        grid=(B, NB),
        in_specs=[
            pl.BlockSpec(memory_space=pltpu.SMEM),
            pl.BlockSpec((1, RB, DF), lambda b, r: (b, r, 0)),
            pl.BlockSpec((1, DF, N), lambda b, r: (b, 0, 0)),
            pl.BlockSpec((1, RB, N), lambda b, r: (b, r, 0)),
        ],
        out_specs=[
            pl.BlockSpec((1, RB, N), lambda b, r: (b, r, 0)),
            pl.BlockSpec((1, RB, 128), lambda b, r: (b, r, 0)),
            pl.BlockSpec((1, RB, 1), lambda b, r: (b * NB + r, 0, 0)),
        ],
        out_shape=[
            jax.ShapeDtypeStruct((B, N, N), jnp.float32),
            jax.ShapeDtypeStruct((B, N, 128), jnp.float32),
            jax.ShapeDtypeStruct((B * NB, RB, 1), jnp.float32),
        ],
    )(scale, xc, xct, q)

    lp_flat, idx_flat = _sc_topk_call(
        lq.reshape(-1), gmax.reshape(-1), tau.reshape(-1)
    )
    lp = lp_flat.reshape(B, N, K)
    idx = idx_flat.reshape(B, N, K)

    offs = (jnp.arange(B, dtype=jnp.int32) * N)[:, None]
    e0 = idx.reshape(B, N * K) + offs
    e1 = jnp.repeat(jnp.arange(N, dtype=jnp.int32), K)[None, :] + offs
    edges_sparse = jnp.stack((e0, e1), 0).reshape(2, -1)
    return xe, edges_sparse, lp


# speedup vs baseline: 15.7642x; 1.1615x over previous
import functools

import jax
import jax.numpy as jnp
from jax import lax
from jax.experimental import pallas as pl
from jax.experimental.pallas import tpu as pltpu
from jax.experimental.pallas import tpu_sc as plsc

B, N, DF, K = 4, 2048, 128, 16
RB = 256
NB = N // RB
BN = B * N

NW = 32
ROWS_W = BN // NW
RG = 16
NGRP = ROWS_W // RG
GCAP = 32
CCAP = 128

NEG = float("-inf")


def _embed_body(x_ref, w_ref, xe_ref, xc_ref, xct_ref):
    x = x_ref[0]
    w = w_ref[...]
    xe = jnp.dot(x, w, preferred_element_type=jnp.float32)
    xe_ref[0] = xe
    mean = jnp.mean(xe, axis=0, keepdims=True)
    xc = xe - mean
    xc_ref[0] = xc
    xct_ref[0] = xc.T


def _lq_body(scale_ref, xcr_ref, xct_ref, q_ref, lq_ref, gmax_ref, tau_ref):
    r0 = pl.program_id(1) * RB
    xr = xcr_ref[0]
    xt = xct_ref[0]
    s = jnp.dot(xr, xt, preferred_element_type=jnp.float32)
    x2r = jnp.sum(xr * xr, axis=1, keepdims=True)
    x2c = jnp.sum(xt * xt, axis=0, keepdims=True)
    d = jnp.maximum(x2r + x2c - 2.0 * s, 0.0)
    scale = scale_ref[0]
    col = lax.broadcasted_iota(jnp.int32, (RB, N), 1)
    row = lax.broadcasted_iota(jnp.int32, (RB, N), 0) + r0
    g = jnp.log(-jnp.log(q_ref[0]))
    vals = jnp.where(col == row, NEG, -d * scale - g)
    lq_ref[...] = vals.reshape(RB // 8, 8, 16, 128).transpose(0, 2, 1, 3)
    gm = vals[:, 0:128]
    for t in range(1, 16):
        gm = jnp.maximum(gm, vals[:, t * 128 : (t + 1) * 128])
    gmax_ref[0] = gm
    g2 = gm
    for _ in range(15):
        m = jnp.max(g2, axis=1, keepdims=True)
        g2 = jnp.where(g2 == m, NEG, g2)
    tau_ref[...] = jnp.max(g2, axis=1, keepdims=True).reshape(RB)


def _sc_topk(lq_hbm, gmax_hbm, tau_hbm, lp_hbm, idx_hbm,
             rowbs, gbs, tbs, lpbs, idxbs, gidb, candv, candp,
             insems, outsems):
    wid = lax.axis_index("s") * 2 + lax.axis_index("c")
    wrow0 = wid * ROWS_W
    lane = lax.iota(jnp.int32, 16)
    ninf = jnp.full((16,), NEG, jnp.float32)

    def fetch(gidx, p):
        row0 = wrow0 + gidx * RG
        pltpu.async_copy(lq_hbm.at[pl.ds(row0 * N, RG * N)], rowbs[p],
                         insems[p])
        pltpu.async_copy(gmax_hbm.at[pl.ds(row0 * 128, RG * 128)], gbs[p],
                         insems[p])
        pltpu.async_copy(tau_hbm.at[pl.ds(row0, RG)], tbs[p], insems[p])

    def drain_in(gidx, p):
        row0 = wrow0 + gidx * RG
        pltpu.make_async_copy(lq_hbm.at[pl.ds(row0 * N, RG * N)], rowbs[p],
                              insems[p]).wait()
        pltpu.make_async_copy(gmax_hbm.at[pl.ds(row0 * 128, RG * 128)],
                              gbs[p], insems[p]).wait()
        pltpu.make_async_copy(tau_hbm.at[pl.ds(row0, RG)], tbs[p],
                              insems[p]).wait()

    def merge_body(m, carry):
        rv, ri, ccs = carry
        valid = (lane + m * 16) < ccs
        c = jnp.where(valid, candv[pl.ds(m * 16, 16)], ninf)
        p = candp[pl.ds(m * 16, 16)]
        cs, cp = plsc.sort_key_val(c, p, descending=True)
        csr = jnp.flip(cs, 0)
        cpr = jnp.flip(cp, 0)
        take = rv >= csr
        nv = jnp.where(take, rv, csr)
        np_ = jnp.where(take, ri, cpr)
        rv, ri = plsc.sort_key_val(nv, np_, descending=True)
        return rv, ri, ccs

    def process(p):
        rowb, gb, tb, lpb, idxb = rowbs[p], gbs[p], tbs[p], lpbs[p], idxbs[p]

        def row_body(i, _):
            tsplat = plsc.load_gather(tb, [jnp.full((16,), i, jnp.int32)])
            ibase = (i // 8) * 16384 + (i % 8) * 128

            @plsc.parallel_loop(0, 8, 1, unroll=4,
                                carry=jnp.zeros((16,), jnp.int32))
            def gcnt(j, cur):
                gm = gb[pl.ds(i * 128 + j * 16, 16)]
                msk = gm >= tsplat
                pos = cur + plsc.cumsum(msk.astype(jnp.int32)) - 1
                msk = msk & (pos < GCAP)
                plsc.store_scatter(gidb, [pos], lane + j * 16, mask=msk)
                return cur + plsc.all_reduce_population_count(msk)

            def round_fn(r, ccur):
                gids = gidb[pl.ds(r * 16, 16)]
                gvalid = (lane + r * 16) < gcnt
                gids = jnp.where(gvalid, gids, 0)

                @plsc.parallel_loop(0, 16, 1, unroll=4, carry=ccur)
                def cc_out(t, cc):
                    colidx = gids + t * 128
                    v = plsc.load_gather(rowb, [gids + (t * 1024 + ibase)],
                                         mask=gvalid)
                    msk = gvalid & (v >= tsplat)
                    pos = cc + plsc.cumsum(msk.astype(jnp.int32)) - 1
                    msk = msk & (pos < CCAP)
                    plsc.store_scatter(candv, [pos], v, mask=msk)
                    plsc.store_scatter(candp, [pos], colidx, mask=msk)
                    return cc + plsc.all_reduce_population_count(msk)
                return cc_out

            ccur = round_fn(0, jnp.zeros((16,), jnp.int32))
            gs = jnp.max(gcnt)
            ccur = lax.cond(gs > 16, lambda c: round_fn(1, c),
                            lambda c: c, ccur)

            nmerge = (jnp.max(ccur) + 15) // 16
            rv, ri, _ = lax.fori_loop(
                0, nmerge, merge_body,
                (ninf, jnp.zeros((16,), jnp.int32), ccur))
            lpb[pl.ds(i * 16, 16)] = rv
            idxb[pl.ds(i * 16, 16)] = ri
            return 0

        lax.fori_loop(0, RG, row_body, 0)

    def put(gidx, p):
        row0 = wrow0 + gidx * RG
        pltpu.async_copy(lpbs[p], lp_hbm.at[pl.ds(row0 * K, RG * K)],
                         outsems[p])
        pltpu.async_copy(idxbs[p], idx_hbm.at[pl.ds(row0 * K, RG * K)],
                         outsems[p])

    def drain_out(gidx, p):
        row0 = wrow0 + gidx * RG
        pltpu.make_async_copy(lpbs[p], lp_hbm.at[pl.ds(row0 * K, RG * K)],
                              outsems[p]).wait()
        pltpu.make_async_copy(idxbs[p], idx_hbm.at[pl.ds(row0 * K, RG * K)],
                              outsems[p]).wait()

    fetch(0, 0)

    def group_pair(g2, _):
        g0 = g2 * 2
        for p in range(2):
            g = g0 + p
            drain_in(g, p)
            nxt = jnp.minimum(g + 1, NGRP - 1)

            @pl.when(g + 1 < NGRP)
            def _():
                fetch(nxt, 1 - p)

            @pl.when(g >= 2)
            def _():
                drain_out(g - 2, p)

            process(p)
            put(g, p)
        return 0

    lax.fori_loop(0, NGRP // 2, group_pair, 0)
    drain_out(NGRP - 2, 0)
    drain_out(NGRP - 1, 1)


_sc_topk_call = functools.partial(
    pl.kernel,
    out_type=[
        jax.ShapeDtypeStruct((BN * K,), jnp.float32),
        jax.ShapeDtypeStruct((BN * K,), jnp.int32),
    ],
    mesh=plsc.VectorSubcoreMesh(core_axis_name="c", subcore_axis_name="s"),
    compiler_params=pltpu.CompilerParams(needs_layout_passes=False),
    scratch_types=[
        [pltpu.VMEM((RG * N,), jnp.float32)] * 2,
        [pltpu.VMEM((RG * 128,), jnp.float32)] * 2,
        [pltpu.VMEM((RG,), jnp.float32)] * 2,
        [pltpu.VMEM((RG * K,), jnp.float32)] * 2,
        [pltpu.VMEM((RG * K,), jnp.int32)] * 2,
        pltpu.VMEM((GCAP,), jnp.int32),
        pltpu.VMEM((CCAP,), jnp.float32),
        pltpu.VMEM((CCAP,), jnp.int32),
        [pltpu.SemaphoreType.DMA] * 2,
        [pltpu.SemaphoreType.DMA] * 2,
    ],
)(_sc_topk)


@jax.jit
def kernel(x, A, W, temperature, q):
    del A
    scale = jnp.exp(jnp.clip(temperature, -4.0, 4.0)).reshape(1)

    xe, xc, xct = pl.pallas_call(
        _embed_body,
        grid=(B,),
        in_specs=[
            pl.BlockSpec((1, N, DF), lambda b: (b, 0, 0)),
            pl.BlockSpec((DF, DF), lambda b: (0, 0)),
        ],
        out_specs=[
            pl.BlockSpec((1, N, DF), lambda b: (b, 0, 0)),
            pl.BlockSpec((1, N, DF), lambda b: (b, 0, 0)),
            pl.BlockSpec((1, DF, N), lambda b: (b, 0, 0)),
        ],
        out_shape=[
            jax.ShapeDtypeStruct((B, N, DF), jnp.float32),
            jax.ShapeDtypeStruct((B, N, DF), jnp.float32),
            jax.ShapeDtypeStruct((B, DF, N), jnp.float32),
        ],
    )(x, W)

    lq, gmax, tau = pl.pallas_call(
        _lq_body,
        grid=(B, NB),
        in_specs=[
            pl.BlockSpec(memory_space=pltpu.SMEM),
            pl.BlockSpec((1, RB, DF), lambda b, r: (b, r, 0)),
            pl.BlockSpec((1, DF, N), lambda b, r: (b, 0, 0)),
            pl.BlockSpec((1, RB, N), lambda b, r: (b, r, 0)),
        ],
        out_specs=[
            pl.BlockSpec((RB // 8, 16, 8, 128),
                         lambda b, r: (b * NB + r, 0, 0, 0)),
            pl.BlockSpec((1, RB, 128), lambda b, r: (b, r, 0)),
            pl.BlockSpec((RB,), lambda b, r: (b * NB + r,)),
        ],
        out_shape=[
            jax.ShapeDtypeStruct((BN // 8, 16, 8, 128), jnp.float32),
            jax.ShapeDtypeStruct((B, N, 128), jnp.float32),
            jax.ShapeDtypeStruct((BN,), jnp.float32),
        ],
    )(scale, xc, xct, q)

    lp_flat, idx_flat = _sc_topk_call(
        lq.reshape(-1), gmax.reshape(-1), tau
    )
    lp = lp_flat.reshape(B, N, K)
    idx = idx_flat.reshape(B, N, K)

    offs = (jnp.arange(B, dtype=jnp.int32) * N)[:, None]
    e0 = idx.reshape(B, N * K) + offs
    e1 = jnp.repeat(jnp.arange(N, dtype=jnp.int32), K)[None, :] + offs
    edges_sparse = jnp.stack((e0, e1), 0).reshape(2, -1)
    return xe, edges_sparse, lp
